# baseline (speedup 1.0000x reference)
import jax
import jax.numpy as jnp
from jax import lax
from jax.experimental import pallas as pl
from jax.experimental.pallas import tpu as pltpu
from jax.experimental.pallas import tpu_sc as plsc

H = 16
L = 2048
TAB = 2 * L - 1
TAB_PAD = 4096
BUF_PAD = TAB_PAD + 16
NC, NS = 2, 16
NW = NC * NS
ROWS_PER_W = (H * L) // NW
GROUPS = ROWS_PER_W // 8
LAG = 4


def _sc_body(bias_hbm, out_hbm, buf, tshift, sem):
    cid = lax.axis_index("c")
    sid = lax.axis_index("s")
    w = cid * NS + sid
    h = w // 2
    i0 = (w % 2) * ROWS_PER_W

    pltpu.sync_copy(bias_hbm.at[h], buf.at[pl.ds(0, TAB_PAD)])

    iota16 = lax.iota(jnp.int32, 16)

    def build(k, carry):
        base = k * 16
        for b in range(8):
            idx = base + (7 - b) + iota16
            tshift[b, pl.ds(base, 16)] = plsc.load_gather(buf, [idx])
        return carry

    lax.fori_loop(0, TAB_PAD // 16, build, 0)

    def emit(g, carry):
        off = pl.multiple_of((L - 8) - i0 - 8 * g, 8)
        row0 = i0 + 8 * g
        pltpu.async_copy(
            tshift.at[:, pl.ds(off, L)],
            out_hbm.at[h, pl.ds(row0, 8), :],
            sem,
        )

        @pl.when(g >= LAG)
        def _():
            pltpu.make_async_copy(
                tshift.at[:, pl.ds(0, L)], out_hbm.at[0, pl.ds(0, 8), :], sem
            ).wait()

        return carry

    lax.fori_loop(0, GROUPS, emit, 0)

    for _ in range(LAG):
        pltpu.make_async_copy(
            tshift.at[:, pl.ds(0, L)], out_hbm.at[0, pl.ds(0, 8), :], sem
        ).wait()


def kernel(bias, length):
    del length
    bias_t = jnp.zeros((H, TAB_PAD), jnp.float32).at[:, :TAB].set(bias.T)

    fn = pl.kernel(
        _sc_body,
        out_type=jax.ShapeDtypeStruct((H, L, L), jnp.float32),
        mesh=plsc.VectorSubcoreMesh(core_axis_name="c", subcore_axis_name="s"),
        scratch_types=[
            pltpu.VMEM((BUF_PAD,), jnp.float32),
            pltpu.VMEM((8, BUF_PAD), jnp.float32),
            pltpu.SemaphoreType.DMA,
        ],
        compiler_params=pltpu.CompilerParams(
            needs_layout_passes=False, use_tc_tiling_on_sc=False
        ),
    )
    return fn(bias_t)


# speedup vs baseline: 2.3085x; 2.3085x over previous
import jax
import jax.numpy as jnp
from jax import lax
from jax.experimental import pallas as pl
from jax.experimental.pallas import tpu as pltpu
from jax.experimental.pallas import tpu_sc as plsc

H = 16
L = 2048
TAB = 2 * L - 1
TAB_PAD = 4096
VW = 4224
BUFW = 4368
NC, NS = 2, 16
NW = NC * NS
ROWS_PER_W = (H * L) // NW
NCLS = 16
KPC = ROWS_PER_W // 8 // NCLS


def _sc_body(bias_hbm, out_hbm, buf, vshift, sem):
    cid = lax.axis_index("c")
    sid = lax.axis_index("s")
    w = cid * NS + sid
    h = w // 2
    i0 = (w % 2) * ROWS_PER_W

    pltpu.sync_copy(bias_hbm.at[h], buf.at[pl.ds(0, TAB_PAD)])

    iota16 = lax.iota(jnp.int32, 16)

    def do_class(c, carry):
        p = c % 2
        r = (120 - 8 * c) % 128

        @pl.when(c >= 2)
        def _():
            for _ in range(KPC):
                pltpu.make_async_copy(
                    vshift.at[0, :, pl.ds(0, L)],
                    out_hbm.at[0, pl.ds(0, 8), :],
                    sem,
                ).wait()

        def build(k, carry2):
            base = k * 16
            for b in range(8):
                idx = base + (r + (7 - b)) + iota16
                vshift[p, b, pl.ds(base, 16)] = plsc.load_gather(buf, [idx])
            return carry2

        lax.fori_loop(0, VW // 16, build, 0)

        for k in range(KPC):
            off128 = pl.multiple_of(1920 - i0 - 128 * k, 128)
            row0 = pl.multiple_of(i0 + 128 * k + 8 * c, 8)
            pltpu.async_copy(
                vshift.at[p, :, pl.ds(off128, L)],
                out_hbm.at[h, pl.ds(row0, 8), :],
                sem,
            )
        return carry

    lax.fori_loop(0, NCLS, do_class, 0)

    for _ in range(2 * KPC):
        pltpu.make_async_copy(
            vshift.at[0, :, pl.ds(0, L)], out_hbm.at[0, pl.ds(0, 8), :], sem
        ).wait()


def kernel(bias, length):
    del length
    bias_t = jnp.zeros((H, TAB_PAD), jnp.float32).at[:, :TAB].set(bias.T)

    fn = pl.kernel(
        _sc_body,
        out_type=jax.ShapeDtypeStruct((H, L, L), jnp.float32),
        mesh=plsc.VectorSubcoreMesh(core_axis_name="c", subcore_axis_name="s"),
        scratch_types=[
            pltpu.VMEM((BUFW,), jnp.float32),
            pltpu.VMEM((2, 8, VW), jnp.float32),
            pltpu.SemaphoreType.DMA,
        ],
        compiler_params=pltpu.CompilerParams(needs_layout_passes=False),
    )
    return fn(bias_t)


# speedup vs baseline: 3.4542x; 1.4963x over previous
import jax
import jax.numpy as jnp
from jax import lax
from jax.experimental import pallas as pl
from jax.experimental.pallas import tpu as pltpu
from jax.experimental.pallas import tpu_sc as plsc

H = 16
L = 2048
TAB = 2 * L - 1
TAB_PAD = 4096
VW = 3968
NC, NS = 2, 16
NW = NC * NS
NCLS = 16
CPW = NCLS // 2
KPC = L // 8 // NCLS


def _sc_body(bias_hbm, out_hbm, buf, vshift, sem):
    cid = lax.axis_index("c")
    sid = lax.axis_index("s")
    w = cid * NS + sid
    h = w // 2
    par = w % 2

    pltpu.sync_copy(bias_hbm.at[h], buf)

    def do_class(cc, carry):
        p = cc % 2
        c = 2 * cc + par
        r = 120 - 8 * c

        @pl.when(cc >= 2)
        def _():
            for _ in range(KPC):
                pltpu.make_async_copy(
                    vshift.at[0, :, pl.ds(0, L)],
                    out_hbm.at[0, pl.ds(0, 8), :],
                    sem,
                ).wait()

        def build(k, carry2):
            base = k * 16
            for b in range(8):
                vshift[p, b, pl.ds(base, 16)] = buf[pl.ds(base + r + (7 - b), 16)]
            return carry2

        lax.fori_loop(0, VW // 16, build, 0, unroll=2)

        for k in range(KPC):
            off128 = pl.multiple_of(1920 - 128 * k, 128)
            row0 = pl.multiple_of(128 * k + 8 * c, 8)
            pltpu.async_copy(
                vshift.at[p, :, pl.ds(off128, L)],
                out_hbm.at[h, pl.ds(row0, 8), :],
                sem,
            )
        return carry

    lax.fori_loop(0, CPW, do_class, 0)

    for _ in range(2 * KPC):
        pltpu.make_async_copy(
            vshift.at[0, :, pl.ds(0, L)], out_hbm.at[0, pl.ds(0, 8), :], sem
        ).wait()


def kernel(bias, length):
    del length
    bias_t = jnp.zeros((H, TAB_PAD), jnp.float32).at[:, :TAB].set(bias.T)

    fn = pl.kernel(
        _sc_body,
        out_type=jax.ShapeDtypeStruct((H, L, L), jnp.float32),
        mesh=plsc.VectorSubcoreMesh(core_axis_name="c", subcore_axis_name="s"),
        scratch_types=[
            pltpu.VMEM((TAB_PAD,), jnp.float32),
            pltpu.VMEM((2, 8, VW), jnp.float32),
            pltpu.SemaphoreType.DMA,
        ],
        compiler_params=pltpu.CompilerParams(needs_layout_passes=False),
    )
    return fn(bias_t)
